# baseline (speedup 1.0000x reference)
import functools

import jax
import jax.numpy as jnp
from jax import lax
from jax.experimental import pallas as pl
from jax.experimental.pallas import tpu as pltpu
from jax.experimental.pallas import tpu_sc as plsc

VOCAB = 100000
EMBED = 32
BATCH = 16384
SEQ = 200

NC, NS = 2, 16
NW = NC * NS
IW = 128
CB = 8
ROWS = CB * SEQ
NSTREAMS = (ROWS + IW - 1) // IW + 1
PROWS = NSTREAMS * IW
NCHUNKS = BATCH // CB
ITERS = NCHUNKS // NW
PAIRS = ITERS // 2


@functools.partial(
    pl.kernel,
    out_type=jax.ShapeDtypeStruct((BATCH, SEQ, EMBED), jnp.float32),
    mesh=plsc.VectorSubcoreMesh(core_axis_name="c", subcore_axis_name="s"),
    scratch_types=[
        pltpu.VMEM((NSTREAMS, IW), jnp.int32),
        pltpu.VMEM((NSTREAMS, IW), jnp.int32),
        pltpu.VMEM((PROWS, EMBED), jnp.float32),
        pltpu.VMEM((PROWS, EMBED), jnp.float32),
        pltpu.SemaphoreType.DMA,
        pltpu.SemaphoreType.DMA,
        pltpu.SemaphoreType.DMA,
        pltpu.SemaphoreType.DMA,
        pltpu.SemaphoreType.DMA,
        pltpu.SemaphoreType.DMA,
    ],
    compiler_params=pltpu.CompilerParams(use_tc_tiling_on_sc=False),
)
def _gather_all(tbl_hbm, idx_hbm, out_hbm,
                idx_v0, idx_v1, rows_v0, rows_v1,
                si0, si1, sg0, sg1, so0, so1):
    idx_v = (idx_v0, idx_v1)
    rows_v = (rows_v0, rows_v1)
    sem_i = (si0, si1)
    sem_g = (sg0, sg1)
    sem_o = (so0, so1)

    wid = lax.axis_index("s") * NC + lax.axis_index("c")
    chunk0 = wid * ITERS

    def idx_src(g):
        return idx_hbm.at[chunk0 + g]

    def fire_writes(g, s):
        b0 = (chunk0 + g) * CB
        for k in range(CB):
            pltpu.async_copy(rows_v[s].at[pl.ds(k * SEQ, SEQ)],
                             out_hbm.at[b0 + k], sem_o[s])

    def drain_writes(s):
        pltpu.make_async_copy(tbl_hbm.at[pl.ds(0, ROWS)],
                              rows_v[s].at[pl.ds(0, ROWS)], sem_o[s]).wait()

    def drain_gathers(s):
        pltpu.make_async_copy(tbl_hbm.at[pl.ds(0, PROWS)],
                              rows_v[s], sem_g[s]).wait()

    pltpu.async_copy(idx_src(0), idx_v[0], sem_i[0])

    def pair(p, carry):
        for s in (0, 1):
            g = p * 2 + s
            pltpu.make_async_copy(idx_src(g), idx_v[s], sem_i[s]).wait()

            @pl.when(g >= 2)
            def _():
                drain_writes(s)

            for j in range(NSTREAMS):
                pltpu.async_copy(tbl_hbm.at[idx_v[s].at[j]],
                                 rows_v[s].at[pl.ds(j * IW, IW)], sem_g[s])

            @pl.when(g >= 1)
            def _():
                drain_gathers(1 - s)
                fire_writes(g - 1, 1 - s)

            @pl.when(g + 1 < ITERS)
            def _():
                pltpu.async_copy(idx_src(g + 1), idx_v[1 - s], sem_i[1 - s])

        return carry

    lax.fori_loop(0, PAIRS, pair, 0)

    sl = (ITERS - 1) % 2
    drain_gathers(sl)
    fire_writes(ITERS - 1, sl)
    drain_writes(1 - sl)
    drain_writes(sl)


def kernel(inchi, table, start_var):
    b, l = inchi.shape
    tok = inchi[:, :-1].astype(jnp.int32)
    idx = jnp.concatenate(
        [jnp.full((b, 1), VOCAB, jnp.int32), tok], axis=1)
    idx_c = idx.reshape(NCHUNKS, ROWS)
    idx_c = jnp.pad(idx_c, ((0, 0), (0, PROWS - ROWS)),
                    constant_values=VOCAB)
    idx_c = idx_c.reshape(NCHUNKS, NSTREAMS, IW)
    tbl = jnp.concatenate([table, start_var], axis=0)
    return _gather_all(tbl, idx_c)


# speedup vs baseline: 2.8477x; 2.8477x over previous
import functools

import jax
import jax.numpy as jnp
from jax import lax
from jax.experimental import pallas as pl
from jax.experimental.pallas import tpu as pltpu
from jax.experimental.pallas import tpu_sc as plsc

VOCAB = 100000
EMBED = 32
BATCH = 16384
SEQ = 200
SPAD = 208

NC, NS = 2, 16
NW = NC * NS
CB = 8
ITERS = BATCH // (NW * CB)
PAIRS = ITERS // 2

SPLITS = ((0, 128), (128, 64), (192, 8))


@functools.partial(
    pl.kernel,
    out_type=jax.ShapeDtypeStruct((BATCH, SEQ, EMBED), jnp.float32),
    mesh=plsc.VectorSubcoreMesh(core_axis_name="c", subcore_axis_name="s"),
    scratch_types=[
        pltpu.VMEM((CB, SEQ), jnp.int32),
        pltpu.VMEM((CB, SEQ), jnp.int32),
        pltpu.VMEM((CB, SPAD), jnp.int32),
        pltpu.VMEM((CB, SPAD), jnp.int32),
        pltpu.VMEM((CB, SEQ, EMBED), jnp.float32),
        pltpu.VMEM((CB, SEQ, EMBED), jnp.float32),
        pltpu.SemaphoreType.DMA,
        pltpu.SemaphoreType.DMA,
        pltpu.SemaphoreType.DMA,
        pltpu.SemaphoreType.DMA,
        pltpu.SemaphoreType.DMA,
        pltpu.SemaphoreType.DMA,
    ],
    compiler_params=pltpu.CompilerParams(use_tc_tiling_on_sc=False,
                                         needs_layout_passes=False),
)
def _embed_all(inchi_hbm, tbl_hbm, out_hbm,
               tok_v0, tok_v1, sidx_v0, sidx_v1, rows_v0, rows_v1,
               si0, si1, sg0, sg1, so0, so1):
    tok_v = (tok_v0, tok_v1)
    sidx_v = (sidx_v0, sidx_v1)
    rows_v = (rows_v0, rows_v1)
    sem_i = (si0, si1)
    sem_g = (sg0, sg1)
    sem_o = (so0, so1)

    wid = lax.axis_index("s") * NC + lax.axis_index("c")
    b0w = wid * ITERS * CB

    lanes = lax.iota(jnp.int32, 16)

    def tok_src(g):
        return inchi_hbm.at[pl.ds(b0w + g * CB, CB)]

    def build_sidx(s):
        for k in range(CB):
            krow = jnp.full((16,), k, jnp.int32)
            for t in range(SEQ // 16 + 1):
                cols = lanes + (t * 16 - 1)
                if t == 0:
                    v = plsc.load_gather(tok_v[s], [krow, jnp.maximum(cols, 0)])
                    v = jnp.where(lanes == 0, VOCAB, v)
                else:
                    v = plsc.load_gather(
                        tok_v[s], [krow, jnp.minimum(cols, SEQ - 1)])
                sidx_v[s][k, pl.ds(t * 16, 16)] = v

    def fire_gathers(s):
        for k in range(CB):
            for (o, n) in SPLITS:
                pltpu.async_copy(tbl_hbm.at[sidx_v[s].at[k, pl.ds(o, n)]],
                                 rows_v[s].at[k, pl.ds(o, n)], sem_g[s])

    def drain_gathers(s):
        for k in range(CB):
            for (o, n) in SPLITS:
                pltpu.make_async_copy(
                    tbl_hbm.at[sidx_v[s].at[k, pl.ds(o, n)]],
                    rows_v[s].at[k, pl.ds(o, n)], sem_g[s]).wait()

    def fire_write(g, s):
        pltpu.async_copy(rows_v[s], out_hbm.at[pl.ds(b0w + g * CB, CB)],
                         sem_o[s])

    def drain_write(g, s):
        pltpu.make_async_copy(rows_v[s], out_hbm.at[pl.ds(b0w + g * CB, CB)],
                              sem_o[s]).wait()

    pltpu.async_copy(tok_src(0), tok_v[0], sem_i[0])

    def pair(p, carry):
        for s in (0, 1):
            g = p * 2 + s
            pltpu.make_async_copy(tok_src(g), tok_v[s], sem_i[s]).wait()
            build_sidx(s)

            @pl.when(g >= 2)
            def _():
                drain_write(g - 2, s)

            fire_gathers(s)

            @pl.when(g >= 1)
            def _():
                drain_gathers(1 - s)
                fire_write(g - 1, 1 - s)

            @pl.when(g + 1 < ITERS)
            def _():
                pltpu.async_copy(tok_src(g + 1), tok_v[1 - s], sem_i[1 - s])

        return carry

    lax.fori_loop(0, PAIRS, pair, 0)

    sl = (ITERS - 1) % 2
    drain_gathers(sl)
    fire_write(ITERS - 1, sl)
    drain_write(ITERS - 2, 1 - sl)
    drain_write(ITERS - 1, sl)


def kernel(inchi, table, start_var):
    tbl = jnp.concatenate([table, start_var], axis=0)
    return _embed_all(inchi.astype(jnp.int32), tbl)
